# Initial kernel scaffold; baseline (speedup 1.0000x reference)
import jax
import jax.numpy as jnp
from jax.experimental import pallas as pl


def kernel(x, table):
    raise NotImplementedError("write your pallas kernel here")


# speedup vs baseline: 4.2983x; 4.2983x over previous
import functools

import jax
import jax.numpy as jnp
from jax import lax
from jax.experimental import pallas as pl
from jax.experimental.pallas import tpu as pltpu
from jax.experimental.pallas import tpu_sc as plsc

NC = 2
NS = 16
L = 16
NW = NC * NS

B = 16384
SEQ = 200
D = 10
N = B * SEQ
PER_W = N // NW
C = 2048
N_CHUNKS = PER_W // C


def _sc_body(x_hbm, tab_hbm, out_hbm, tab_v, xb, ob):
    wid = lax.axis_index("s") * NC + lax.axis_index("c")
    base = wid * PER_W

    pltpu.sync_copy(tab_hbm, tab_v.at[pl.ds(0, 100 * D)])
    tab_v[pl.ds(100 * D, L)] = jnp.zeros((L,), jnp.float32)
    iota = lax.iota(jnp.int32, L)
    iota_d = iota * D

    @pl.loop(0, N_CHUNKS)
    def _chunk(s):
        start = base + s * C
        pltpu.sync_copy(x_hbm.at[pl.ds(start, C)], xb)

        @pl.loop(0, C // L)
        def _group(g):
            xg = xb[pl.ds(g * L, L)]
            a = jnp.where(xg == 0, jnp.int32(100), xg) * D
            sidx = iota_d + g * (L * D)
            for j in range(D):
                v = plsc.load_gather(tab_v, [a + j])
                plsc.store_scatter(ob, [sidx + j], v)

        pltpu.sync_copy(ob, out_hbm.at[pl.ds(start * D, C * D)])


@jax.jit
def kernel(x, table):
    xf = x.reshape(-1).astype(jnp.int32)
    tf = table.reshape(-1)
    call = pl.kernel(
        _sc_body,
        out_type=jax.ShapeDtypeStruct((N * D,), jnp.float32),
        mesh=plsc.VectorSubcoreMesh(core_axis_name="c", subcore_axis_name="s",
                                    num_cores=NC, num_subcores=NS),
        compiler_params=pltpu.CompilerParams(needs_layout_passes=False),
        scratch_types=[
            pltpu.VMEM((100 * D + L,), jnp.float32),
            pltpu.VMEM((C,), jnp.int32),
            pltpu.VMEM((C * D,), jnp.float32),
        ],
    )
    out = call(xf, tf)
    return out.reshape(B, SEQ, D)


# speedup vs baseline: 33.2582x; 7.7376x over previous
import jax
import jax.numpy as jnp
from jax import lax
from jax.experimental import pallas as pl
from jax.experimental.pallas import tpu as pltpu
from jax.experimental.pallas import tpu_sc as plsc

NC = 2
NS = 16
L = 16
NW = NC * NS

B = 16384
SEQ = 200
D = 10
CB = B // NW
LB = 8
N_CHUNKS = SEQ // LB
BG = CB // L


def _sc_body(xt_hbm, tab_hbm, ot_hbm, tab_v, xb, ob):
    wid = lax.axis_index("s") * NC + lax.axis_index("c")
    b0 = wid * CB

    pltpu.sync_copy(tab_hbm, tab_v.at[pl.ds(0, 100 * D)])
    tab_v[pl.ds(100 * D, L)] = jnp.zeros((L,), jnp.float32)

    @pl.loop(0, N_CHUNKS)
    def _chunk(s):
        l0 = s * LB
        pltpu.sync_copy(xt_hbm.at[pl.ds(l0, LB), pl.ds(b0, CB)], xb)

        for ll in range(LB):
            @pl.loop(0, BG)
            def _group(g, ll=ll):
                xg = xb[ll, pl.ds(g * L, L)]
                a = jnp.where(xg == 0, jnp.int32(100), xg) * D
                for j in range(D):
                    v = plsc.load_gather(tab_v, [a + j])
                    ob[j, ll, pl.ds(g * L, L)] = v

        pltpu.sync_copy(ob, ot_hbm.at[:, pl.ds(l0, LB), pl.ds(b0, CB)])


@jax.jit
def kernel(x, table):
    xt = x.T
    tf = table.reshape(-1)
    call = pl.kernel(
        _sc_body,
        out_type=jax.ShapeDtypeStruct((D, SEQ, B), jnp.float32),
        mesh=plsc.VectorSubcoreMesh(core_axis_name="c", subcore_axis_name="s",
                                    num_cores=NC, num_subcores=NS),
        compiler_params=pltpu.CompilerParams(needs_layout_passes=False),
        scratch_types=[
            pltpu.VMEM((100 * D + L,), jnp.float32),
            pltpu.VMEM((LB, CB), jnp.int32),
            pltpu.VMEM((D, LB, CB), jnp.float32),
        ],
    )
    ot = call(xt, tf)
    return ot.transpose(2, 1, 0)
